# baseline (speedup 1.0000x reference)
import jax
import jax.numpy as jnp
from jax.experimental import pallas as pl

_B = 512
_L = 16
_P = 240
_H = 768
_PB = 8


def _fused_body(pooled_ref, pairs_ref, w_ref, b_ref, eps_ref, out_ref):
    x = pooled_ref[...]
    w = w_ref[...]
    eps = eps_ref[0, 0]
    bias = b_ref[0, 0]
    logits = jnp.sum(x * w, axis=1) + bias
    val = jax.nn.sigmoid(logits).reshape(_PB, _P) + eps

    pairs = pairs_ref[...]
    idx = pairs[:, :, 0] * _L + pairs[:, :, 1]

    lane = jax.lax.broadcasted_iota(jnp.int32, (_PB, _L * _L), 1)
    acc = jnp.full((_PB, _L * _L), 1.0 + eps, dtype=jnp.float32)
    for k in range(_P):
        acc = jnp.where(idx[:, k:k + 1] == lane, val[:, k:k + 1], acc)
    out_ref[...] = acc


def kernel(pooled_output, pairs_list, passage_length, pairs_num, W, b, epsilon):
    del passage_length, pairs_num
    pairs3 = pairs_list.reshape(_B, _P, 2)
    b2 = jnp.reshape(b, (1, 1)).astype(jnp.float32)
    eps2 = jnp.reshape(epsilon, (1, 1)).astype(jnp.float32)

    grid = (_B // _PB,)
    adj = pl.pallas_call(
        _fused_body,
        grid=grid,
        in_specs=[
            pl.BlockSpec((_PB * _P, _H), lambda i: (i, 0)),
            pl.BlockSpec((_PB, _P, 2), lambda i: (i, 0, 0)),
            pl.BlockSpec((1, _H), lambda i: (0, 0)),
            pl.BlockSpec((1, 1), lambda i: (0, 0)),
            pl.BlockSpec((1, 1), lambda i: (0, 0)),
        ],
        out_specs=pl.BlockSpec((_PB, _L * _L), lambda i: (i, 0)),
        out_shape=jax.ShapeDtypeStruct((_B, _L * _L), jnp.float32),
    )(pooled_output, pairs3, W, b2, eps2)
    return adj.reshape(_B, _L, _L)


# speedup vs baseline: 5.2717x; 5.2717x over previous
import jax
import jax.numpy as jnp
from jax import lax
from jax.experimental import pallas as pl

_B = 512
_L = 16
_P = 240
_H = 768
_S = _L * _L
_PB = 8


def _fused_body(x_ref, p0_ref, p1_ref, w_ref, b_ref, eps_ref, out_ref):
    x = x_ref[...]
    w = w_ref[...]
    eps = eps_ref[0, 0]
    bias = b_ref[0, 0]
    logits = jnp.sum(x * w[None, :, :], axis=2, keepdims=True) + bias
    val = jax.nn.sigmoid(logits)
    idx = p0_ref[...] * _L + p1_ref[...]
    k3 = lax.broadcasted_iota(jnp.int32, (_PB, _P, 1), 1).astype(jnp.float32)
    packed = k3 + val
    lane = lax.broadcasted_iota(jnp.int32, (_PB, _P, _S), 2)
    comb = jnp.where(idx == lane, packed, -1.0)
    red = jnp.max(comb, axis=1)
    frac = red - jnp.floor(red)
    out_ref[...] = jnp.where(red >= 0.0, frac, 1.0) + eps


def kernel(pooled_output, pairs_list, passage_length, pairs_num, W, b, epsilon):
    del passage_length, pairs_num
    x3 = pooled_output.reshape(_B, _P, _H)
    p0 = pairs_list[:, 0].reshape(_B, _P, 1)
    p1 = pairs_list[:, 1].reshape(_B, _P, 1)
    b2 = jnp.reshape(b, (1, 1)).astype(jnp.float32)
    eps2 = jnp.reshape(epsilon, (1, 1)).astype(jnp.float32)

    grid = (_B // _PB,)
    adj = pl.pallas_call(
        _fused_body,
        grid=grid,
        in_specs=[
            pl.BlockSpec((_PB, _P, _H), lambda i: (i, 0, 0)),
            pl.BlockSpec((_PB, _P, 1), lambda i: (i, 0, 0)),
            pl.BlockSpec((_PB, _P, 1), lambda i: (i, 0, 0)),
            pl.BlockSpec((1, _H), lambda i: (0, 0)),
            pl.BlockSpec((1, 1), lambda i: (0, 0)),
            pl.BlockSpec((1, 1), lambda i: (0, 0)),
        ],
        out_specs=pl.BlockSpec((_PB, _S), lambda i: (i, 0)),
        out_shape=jax.ShapeDtypeStruct((_B, _S), jnp.float32),
    )(x3, p0, p1, W, b2, eps2)
    return adj.reshape(_B, _L, _L)
